# initial kernel scaffold (unmeasured)
import jax
import jax.numpy as jnp
from jax import lax
from jax.experimental import pallas as pl
from jax.experimental.pallas import tpu as pltpu

B, S, H, Dh, Dr = 2, 512, 16, 128, 32
D = 2048
DC = 128
BS = B * S
SCALE = (Dh + Dr) ** -0.5


def _mm(a, b):
    return lax.dot_general(a, b, (((1,), (0,)), ((), ())),
                           preferred_element_type=jnp.float32)


def _mm_t(a, b):
    return lax.dot_general(a, b, (((1,), (1,)), ((), ())),
                           preferred_element_type=jnp.float32)


def _kv_body(x_ref, wdkv_ref, wuk_ref, wuv_ref,
             k_ref, v_ref, xbf_ref,
             c_mine, c_other, wuk_mine, wuk_other, wuv_mine, wuv_other,
             send_sems, recv_sems):
    my_x = lax.axis_index("x")
    my_y = lax.axis_index("y")
    peer = (my_x, 1 - my_y)

    barrier = pltpu.get_barrier_semaphore()
    pl.semaphore_signal(barrier, inc=1, device_id=peer,
                        device_id_type=pl.DeviceIdType.MESH)
    pl.semaphore_wait(barrier, 1)

    xbf_ref[...] = x_ref[...].astype(jnp.bfloat16)
    wuk_mine[...] = wuk_ref[...].astype(jnp.bfloat16)
    wuv_mine[...] = wuv_ref[...].astype(jnp.bfloat16)
    c_mine[...] = _mm(xbf_ref[...],
                      wdkv_ref[...].astype(jnp.bfloat16)).astype(jnp.bfloat16)

    copies = []
    for i, (src, dst) in enumerate([(c_mine, c_other),
                                    (wuk_mine, wuk_other),
                                    (wuv_mine, wuv_other)]):
        rdma = pltpu.make_async_remote_copy(
            src_ref=src, dst_ref=dst,
            send_sem=send_sems.at[i], recv_sem=recv_sems.at[i],
            device_id=peer, device_id_type=pl.DeviceIdType.MESH)
        rdma.start()
        copies.append(rdma)
    for rdma in copies:
        rdma.wait()

    k = _mm(c_mine[...], wuk_mine[...]) + _mm(c_other[...], wuk_other[...])
    k_ref[...] = k.astype(jnp.bfloat16)
    v = _mm(c_mine[...], wuv_mine[...]) + _mm(c_other[...], wuv_other[...])
    v_ref[...] = v.astype(jnp.bfloat16)


def _q_body(xbf_ref, wq_ref, wqr_ref, wkr_ref, q_ref, qr_ref, kr_ref):
    xb = xbf_ref[...]
    q_ref[...] = _mm(xb, wq_ref[...].astype(jnp.bfloat16)).astype(jnp.bfloat16)
    qr_ref[...] = _mm(xb, wqr_ref[...].astype(jnp.bfloat16)).astype(jnp.bfloat16)
    kr_ref[...] = _mm(xb, wkr_ref[...].astype(jnp.bfloat16)).astype(jnp.bfloat16)


def _attn_body(q_ref, k_ref, v_ref, qr_ref, kr_ref, wo_ref, o_ref):
    h = pl.program_id(1)
    s = _mm_t(q_ref[...], k_ref[...])
    s += _mm_t(qr_ref[...], kr_ref[...])
    s *= SCALE
    m = jnp.max(s, axis=1, keepdims=True)
    p = jnp.exp(s - m)
    p = p / jnp.sum(p, axis=1, keepdims=True)
    o_bh = _mm(p.astype(jnp.bfloat16), v_ref[...]).astype(jnp.bfloat16)
    contrib = _mm(o_bh, wo_ref[...].astype(jnp.bfloat16))

    @pl.when(h == 0)
    def _():
        o_ref[...] = contrib

    @pl.when(h != 0)
    def _():
        o_ref[...] += contrib


def kernel(x, Wdkv, Wuk, Wuv, Wq, Wqr, Wkr, Wo):
    x2 = x.reshape(BS, D)

    k, v, xbf = pl.pallas_call(
        _kv_body,
        out_shape=[
            jax.ShapeDtypeStruct((BS, D), jnp.bfloat16),
            jax.ShapeDtypeStruct((BS, D), jnp.bfloat16),
            jax.ShapeDtypeStruct((BS, D), jnp.bfloat16),
        ],
        in_specs=[pl.BlockSpec(memory_space=pltpu.VMEM)] * 4,
        out_specs=[pl.BlockSpec(memory_space=pltpu.VMEM)] * 3,
        scratch_shapes=[
            pltpu.VMEM((BS, DC), jnp.bfloat16),
            pltpu.VMEM((BS, DC), jnp.bfloat16),
            pltpu.VMEM((DC, D), jnp.bfloat16),
            pltpu.VMEM((DC, D), jnp.bfloat16),
            pltpu.VMEM((DC, D), jnp.bfloat16),
            pltpu.VMEM((DC, D), jnp.bfloat16),
            pltpu.SemaphoreType.DMA((3,)),
            pltpu.SemaphoreType.DMA((3,)),
        ],
        compiler_params=pltpu.CompilerParams(collective_id=0),
    )(x2, Wdkv, Wuk, Wuv)

    q, qr, kr = pl.pallas_call(
        _q_body,
        out_shape=[
            jax.ShapeDtypeStruct((BS, D), jnp.bfloat16),
            jax.ShapeDtypeStruct((BS, H * Dr), jnp.bfloat16),
            jax.ShapeDtypeStruct((BS, Dr), jnp.bfloat16),
        ],
        in_specs=[pl.BlockSpec(memory_space=pltpu.VMEM)] * 4,
        out_specs=[pl.BlockSpec(memory_space=pltpu.VMEM)] * 3,
    )(xbf, Wq, Wqr, Wkr)

    out = pl.pallas_call(
        _attn_body,
        grid=(B, H),
        out_shape=jax.ShapeDtypeStruct((BS, D), jnp.float32),
        in_specs=[
            pl.BlockSpec((S, Dh), lambda b, h: (b, h)),
            pl.BlockSpec((S, Dh), lambda b, h: (b, h)),
            pl.BlockSpec((S, Dh), lambda b, h: (b, h)),
            pl.BlockSpec((S, Dr), lambda b, h: (b, h)),
            pl.BlockSpec((S, Dr), lambda b, h: (b, 0)),
            pl.BlockSpec((Dh, D), lambda b, h: (h, 0)),
        ],
        out_specs=pl.BlockSpec((S, D), lambda b, h: (b, 0)),
    )(q, k, v, qr, kr, Wo)

    return out.reshape(B, S, D)


# baseline (device time: 128221 ns/iter reference)
import jax
import jax.numpy as jnp
from jax import lax
from jax.experimental import pallas as pl
from jax.experimental.pallas import tpu as pltpu

B, S, H, Dh, Dr = 2, 512, 16, 128, 32
D = 2048
DC = 128
BS = B * S
SCALE = (Dh + Dr) ** -0.5


def _mm(a, b):
    return lax.dot_general(a, b, (((1,), (0,)), ((), ())),
                           preferred_element_type=jnp.float32)


def _mm_t(a, b):
    return lax.dot_general(a, b, (((1,), (1,)), ((), ())),
                           preferred_element_type=jnp.float32)


def _kv_body(x_ref, wdkv_ref, wuk_ref, wuv_ref,
             k_ref, v_ref, xbf_ref,
             c_mine, c_other, wuk_mine, wuk_other, wuv_mine, wuv_other,
             send_sems, recv_sems):
    my_x = lax.axis_index("x")
    my_y = lax.axis_index("y")
    peer = (my_x, 1 - my_y)

    barrier = pltpu.get_barrier_semaphore()
    pl.semaphore_signal(barrier, inc=1, device_id=peer,
                        device_id_type=pl.DeviceIdType.MESH)
    pl.semaphore_wait(barrier, 1)

    xbf_ref[...] = x_ref[...].astype(jnp.bfloat16)
    wuk_mine[...] = wuk_ref[...].astype(jnp.bfloat16)
    wuv_mine[...] = wuv_ref[...].astype(jnp.bfloat16)
    c_mine[...] = _mm(xbf_ref[...],
                      wdkv_ref[...].astype(jnp.bfloat16)).astype(jnp.bfloat16)

    copies = []
    for i, (src, dst) in enumerate([(c_mine, c_other),
                                    (wuk_mine, wuk_other),
                                    (wuv_mine, wuv_other)]):
        rdma = pltpu.make_async_remote_copy(
            src_ref=src, dst_ref=dst,
            send_sem=send_sems.at[i], recv_sem=recv_sems.at[i],
            device_id=peer, device_id_type=pl.DeviceIdType.MESH)
        rdma.start()
        copies.append(rdma)
    for rdma in copies:
        rdma.wait()

    k = _mm(c_mine[...], wuk_mine[...]) + _mm(c_other[...], wuk_other[...])
    k_ref[...] = k.astype(jnp.bfloat16)
    v = _mm(c_mine[...], wuv_mine[...]) + _mm(c_other[...], wuv_other[...])
    v_ref[...] = v.astype(jnp.bfloat16)


def _q_body(xbf_ref, wq_ref, wqr_ref, wkr_ref, q_ref, qr_ref, kr_ref):
    xb = xbf_ref[...]
    q_ref[...] = _mm(xb, wq_ref[...].astype(jnp.bfloat16)).astype(jnp.bfloat16)
    qr = _mm(xb, wqr_ref[...].astype(jnp.bfloat16)).astype(jnp.bfloat16)
    for b in range(B):
        for h in range(H):
            qr_ref[(b * H + h) * S:(b * H + h + 1) * S, :] = (
                qr[b * S:(b + 1) * S, h * Dr:(h + 1) * Dr])
    kr_ref[...] = _mm(xb, wkr_ref[...].astype(jnp.bfloat16)).astype(jnp.bfloat16)


def _attn_body(q_ref, k_ref, v_ref, qr_ref, kr_ref, wo_ref, o_ref):
    h = pl.program_id(1)
    s = _mm_t(q_ref[...], k_ref[...])
    s += _mm_t(qr_ref[...], kr_ref[...])
    s *= SCALE
    m = jnp.max(s, axis=1, keepdims=True)
    p = jnp.exp(s - m)
    p = p / jnp.sum(p, axis=1, keepdims=True)
    o_bh = _mm(p.astype(jnp.bfloat16), v_ref[...]).astype(jnp.bfloat16)
    contrib = _mm(o_bh, wo_ref[...].astype(jnp.bfloat16))

    @pl.when(h == 0)
    def _():
        o_ref[...] = contrib

    @pl.when(h != 0)
    def _():
        o_ref[...] += contrib


def kernel(x, Wdkv, Wuk, Wuv, Wq, Wqr, Wkr, Wo):
    x2 = x.reshape(BS, D)

    k, v, xbf = pl.pallas_call(
        _kv_body,
        out_shape=[
            jax.ShapeDtypeStruct((BS, D), jnp.bfloat16),
            jax.ShapeDtypeStruct((BS, D), jnp.bfloat16),
            jax.ShapeDtypeStruct((BS, D), jnp.bfloat16),
        ],
        in_specs=[pl.BlockSpec(memory_space=pltpu.VMEM)] * 4,
        out_specs=[pl.BlockSpec(memory_space=pltpu.VMEM)] * 3,
        scratch_shapes=[
            pltpu.VMEM((BS, DC), jnp.bfloat16),
            pltpu.VMEM((BS, DC), jnp.bfloat16),
            pltpu.VMEM((DC, D), jnp.bfloat16),
            pltpu.VMEM((DC, D), jnp.bfloat16),
            pltpu.VMEM((DC, D), jnp.bfloat16),
            pltpu.VMEM((DC, D), jnp.bfloat16),
            pltpu.SemaphoreType.DMA((3,)),
            pltpu.SemaphoreType.DMA((3,)),
        ],
        compiler_params=pltpu.CompilerParams(collective_id=0),
    )(x2, Wdkv, Wuk, Wuv)

    q, qr, kr = pl.pallas_call(
        _q_body,
        out_shape=[
            jax.ShapeDtypeStruct((BS, D), jnp.bfloat16),
            jax.ShapeDtypeStruct((B * H * S, Dr), jnp.bfloat16),
            jax.ShapeDtypeStruct((BS, Dr), jnp.bfloat16),
        ],
        in_specs=[pl.BlockSpec(memory_space=pltpu.VMEM)] * 4,
        out_specs=[pl.BlockSpec(memory_space=pltpu.VMEM)] * 3,
    )(xbf, Wq, Wqr, Wkr)

    out = pl.pallas_call(
        _attn_body,
        grid=(B, H),
        out_shape=jax.ShapeDtypeStruct((BS, D), jnp.float32),
        in_specs=[
            pl.BlockSpec((S, Dh), lambda b, h: (b, h)),
            pl.BlockSpec((S, Dh), lambda b, h: (b, h)),
            pl.BlockSpec((S, Dh), lambda b, h: (b, h)),
            pl.BlockSpec((S, Dr), lambda b, h: (b * H + h, 0)),
            pl.BlockSpec((S, Dr), lambda b, h: (b, 0)),
            pl.BlockSpec((Dh, D), lambda b, h: (h, 0)),
        ],
        out_specs=pl.BlockSpec((S, D), lambda b, h: (b, 0)),
    )(q, k, v, qr, kr, Wo)

    return out.reshape(B, S, D)


# device time: 95475 ns/iter; 1.3430x vs baseline; 1.3430x over previous
import jax
import jax.numpy as jnp
from jax import lax
from jax.experimental import pallas as pl
from jax.experimental.pallas import tpu as pltpu

B, S, H, Dh, Dr = 2, 512, 16, 128, 32
D = 2048
DC = 128
BS = B * S
SCALE = (Dh + Dr) ** -0.5


def _mm(a, b):
    return lax.dot_general(a, b, (((1,), (0,)), ((), ())),
                           preferred_element_type=jnp.float32)


def _mm_t(a, b):
    return lax.dot_general(a, b, (((1,), (1,)), ((), ())),
                           preferred_element_type=jnp.float32)


def _cast_body(x_ref, xbf_ref):
    xbf_ref[...] = x_ref[...].astype(jnp.bfloat16)


def _proj_body(xbf_ref, wdkv_ref, wuk_ref, wuv_ref, wq_ref, wqr_ref, wkr_ref,
               q_ref, qr_ref, kr_ref, k_ref, v_ref,
               c_mine, c_other, wuk_mine, wuk_other, wuv_mine, wuv_other,
               send_sems, recv_sems):
    my_x = lax.axis_index("x")
    my_y = lax.axis_index("y")
    peer = (my_x, 1 - my_y)

    barrier = pltpu.get_barrier_semaphore()
    pl.semaphore_signal(barrier, inc=1, device_id=peer,
                        device_id_type=pl.DeviceIdType.MESH)

    xb = xbf_ref[...]
    wuk_mine[...] = wuk_ref[...].astype(jnp.bfloat16)
    wuv_mine[...] = wuv_ref[...].astype(jnp.bfloat16)
    c_mine[...] = _mm(xb, wdkv_ref[...].astype(jnp.bfloat16)).astype(jnp.bfloat16)

    pl.semaphore_wait(barrier, 1)
    copies = []
    for i, (src, dst) in enumerate([(c_mine, c_other),
                                    (wuk_mine, wuk_other),
                                    (wuv_mine, wuv_other)]):
        rdma = pltpu.make_async_remote_copy(
            src_ref=src, dst_ref=dst,
            send_sem=send_sems.at[i], recv_sem=recv_sems.at[i],
            device_id=peer, device_id_type=pl.DeviceIdType.MESH)
        rdma.start()
        copies.append(rdma)

    half = D // 2
    for j in range(2):
        q_ref[:, j * half:(j + 1) * half] = _mm(
            xb, (wq_ref[:, j * half:(j + 1) * half] * SCALE).astype(jnp.bfloat16)
        ).astype(jnp.bfloat16)
    qr = _mm(xb, (wqr_ref[...] * SCALE).astype(jnp.bfloat16)).astype(jnp.bfloat16)
    for b in range(B):
        for h in range(H):
            qr_ref[(b * H + h) * S:(b * H + h + 1) * S, :] = (
                qr[b * S:(b + 1) * S, h * Dr:(h + 1) * Dr])
    kr_ref[...] = _mm(xb, wkr_ref[...].astype(jnp.bfloat16)).astype(jnp.bfloat16)

    for rdma in copies:
        rdma.wait()

    for j in range(2):
        k_ref[:, j * half:(j + 1) * half] = (
            _mm(c_mine[...], wuk_mine[:, j * half:(j + 1) * half])
            + _mm(c_other[...], wuk_other[:, j * half:(j + 1) * half])
        ).astype(jnp.bfloat16)
        v_ref[:, j * half:(j + 1) * half] = (
            _mm(c_mine[...], wuv_mine[:, j * half:(j + 1) * half])
            + _mm(c_other[...], wuv_other[:, j * half:(j + 1) * half])
        ).astype(jnp.bfloat16)


def _attn_body(q_ref, k_ref, v_ref, qr_ref, kr_ref, o_ref):
    s = _mm_t(q_ref[...], k_ref[...])
    s += _mm_t(qr_ref[...], kr_ref[...])
    p = jnp.exp(s).astype(jnp.bfloat16)
    rs = jnp.sum(p.astype(jnp.float32), axis=1, keepdims=True)
    o = _mm(p, v_ref[...])
    o_ref[...] = (o * (1.0 / rs)).astype(jnp.bfloat16)


def _out_body(o_ref, wo_ref, out_ref):
    half = D // 2
    for j in range(2):
        out_ref[:, j * half:(j + 1) * half] = _mm(
            o_ref[...], wo_ref[:, j * half:(j + 1) * half].astype(jnp.bfloat16))


def kernel(x, Wdkv, Wuk, Wuv, Wq, Wqr, Wkr, Wo):
    x2 = x.reshape(BS, D)

    xbf = pl.pallas_call(
        _cast_body,
        out_shape=jax.ShapeDtypeStruct((BS, D), jnp.bfloat16),
        in_specs=[pl.BlockSpec(memory_space=pltpu.VMEM)],
        out_specs=pl.BlockSpec(memory_space=pltpu.VMEM),
    )(x2)

    q, qr, kr, k, v = pl.pallas_call(
        _proj_body,
        out_shape=[
            jax.ShapeDtypeStruct((BS, D), jnp.bfloat16),
            jax.ShapeDtypeStruct((B * H * S, Dr), jnp.bfloat16),
            jax.ShapeDtypeStruct((BS, Dr), jnp.bfloat16),
            jax.ShapeDtypeStruct((BS, D), jnp.bfloat16),
            jax.ShapeDtypeStruct((BS, D), jnp.bfloat16),
        ],
        in_specs=[pl.BlockSpec(memory_space=pltpu.VMEM)] * 7,
        out_specs=[pl.BlockSpec(memory_space=pltpu.VMEM)] * 5,
        scratch_shapes=[
            pltpu.VMEM((BS, DC), jnp.bfloat16),
            pltpu.VMEM((BS, DC), jnp.bfloat16),
            pltpu.VMEM((DC, D), jnp.bfloat16),
            pltpu.VMEM((DC, D), jnp.bfloat16),
            pltpu.VMEM((DC, D), jnp.bfloat16),
            pltpu.VMEM((DC, D), jnp.bfloat16),
            pltpu.SemaphoreType.DMA((3,)),
            pltpu.SemaphoreType.DMA((3,)),
        ],
        compiler_params=pltpu.CompilerParams(collective_id=0),
    )(xbf, Wdkv, Wuk, Wuv, Wq, Wqr, Wkr)

    o = pl.pallas_call(
        _attn_body,
        grid=(B, H),
        out_shape=jax.ShapeDtypeStruct((BS, D), jnp.bfloat16),
        in_specs=[
            pl.BlockSpec((S, Dh), lambda b, h: (b, h)),
            pl.BlockSpec((S, Dh), lambda b, h: (b, h)),
            pl.BlockSpec((S, Dh), lambda b, h: (b, h)),
            pl.BlockSpec((S, Dr), lambda b, h: (b * H + h, 0)),
            pl.BlockSpec((S, Dr), lambda b, h: (b, 0)),
        ],
        out_specs=pl.BlockSpec((S, Dh), lambda b, h: (b, h)),
    )(q, k, v, qr, kr)

    out = pl.pallas_call(
        _out_body,
        out_shape=jax.ShapeDtypeStruct((BS, D), jnp.float32),
        in_specs=[pl.BlockSpec(memory_space=pltpu.VMEM)] * 2,
        out_specs=pl.BlockSpec(memory_space=pltpu.VMEM),
    )(o, Wo)

    return out.reshape(B, S, D)


# device time: 84079 ns/iter; 1.5250x vs baseline; 1.1355x over previous
import jax
import jax.numpy as jnp
from jax import lax
from jax.experimental import pallas as pl
from jax.experimental.pallas import tpu as pltpu

B, S, H, Dh, Dr = 2, 512, 16, 128, 32
D = 2048
DC = 128
BS = B * S
SCALE = (Dh + Dr) ** -0.5


def _mm(a, b):
    return lax.dot_general(a, b, (((1,), (0,)), ((), ())),
                           preferred_element_type=jnp.float32)


def _mm_t(a, b):
    return lax.dot_general(a, b, (((1,), (1,)), ((), ())),
                           preferred_element_type=jnp.float32)


def _cast_body(x_ref, xbf_ref):
    xbf_ref[...] = x_ref[...].astype(jnp.bfloat16)


def _proj_body(xbf_ref, wdkv_ref, wuk_ref, wuv_ref, wq_ref, wqr_ref, wkr_ref,
               q_ref, qr_ref, kr_ref, k_ref, v_ref,
               c_mine, c_other, wuk_mine, wuk_other, wuv_mine, wuv_other,
               send_sems, recv_sems):
    my_x = lax.axis_index("x")
    my_y = lax.axis_index("y")
    peer = (my_x, 1 - my_y)

    barrier = pltpu.get_barrier_semaphore()
    pl.semaphore_signal(barrier, inc=1, device_id=peer,
                        device_id_type=pl.DeviceIdType.MESH)

    xb = xbf_ref[...]
    wuk_mine[...] = wuk_ref[...].astype(jnp.bfloat16)
    wuv_mine[...] = wuv_ref[...].astype(jnp.bfloat16)
    c_mine[...] = _mm(xb, wdkv_ref[...].astype(jnp.bfloat16)).astype(jnp.bfloat16)

    pl.semaphore_wait(barrier, 1)
    copies = []
    for i, (src, dst) in enumerate([(c_mine, c_other),
                                    (wuk_mine, wuk_other),
                                    (wuv_mine, wuv_other)]):
        rdma = pltpu.make_async_remote_copy(
            src_ref=src, dst_ref=dst,
            send_sem=send_sems.at[i], recv_sem=recv_sems.at[i],
            device_id=peer, device_id_type=pl.DeviceIdType.MESH)
        rdma.start()
        copies.append(rdma)

    half = D // 2
    for j in range(2):
        q_ref[:, j * half:(j + 1) * half] = _mm(
            xb, (wq_ref[:, j * half:(j + 1) * half] * SCALE).astype(jnp.bfloat16)
        ).astype(jnp.bfloat16)
    qr = _mm(xb, (wqr_ref[...] * SCALE).astype(jnp.bfloat16)).astype(jnp.bfloat16)
    for b in range(B):
        for h in range(H):
            qr_ref[(b * H + h) * S:(b * H + h + 1) * S, :] = (
                qr[b * S:(b + 1) * S, h * Dr:(h + 1) * Dr])
    kr_ref[...] = _mm(xb, wkr_ref[...].astype(jnp.bfloat16)).astype(jnp.bfloat16)

    for rdma in copies:
        rdma.wait()

    for j in range(2):
        k_ref[:, j * half:(j + 1) * half] = (
            _mm(c_mine[...], wuk_mine[:, j * half:(j + 1) * half])
            + _mm(c_other[...], wuk_other[:, j * half:(j + 1) * half])
        ).astype(jnp.bfloat16)
        v_ref[:, j * half:(j + 1) * half] = (
            _mm(c_mine[...], wuv_mine[:, j * half:(j + 1) * half])
            + _mm(c_other[...], wuv_other[:, j * half:(j + 1) * half])
        ).astype(jnp.bfloat16)


def _attn_out_body(q_ref, k_ref, v_ref, qr_ref, kr_ref, wo_ref, out_ref,
                   o_acc, wo_bf):
    b = pl.program_id(0)

    @pl.when(b == 0)
    def _():
        wo_bf[...] = wo_ref[...].astype(jnp.bfloat16)

    kr = kr_ref[...]
    for h in range(H):
        hs = slice(h * Dh, (h + 1) * Dh)
        s = _mm_t(q_ref[:, hs], k_ref[:, hs])
        s += _mm_t(qr_ref[h * S:(h + 1) * S, :], kr)
        p = jnp.exp(s)
        rs = jnp.sum(p, axis=1, keepdims=True)
        o = _mm(p, v_ref[:, hs].astype(jnp.float32))
        o_acc[:, hs] = (o * (1.0 / rs)).astype(jnp.bfloat16)

    half = D // 2
    for j in range(2):
        js = slice(j * half, (j + 1) * half)
        out_ref[:, js] = _mm(o_acc[...], wo_bf[:, js])


def kernel(x, Wdkv, Wuk, Wuv, Wq, Wqr, Wkr, Wo):
    x2 = x.reshape(BS, D)

    xbf = pl.pallas_call(
        _cast_body,
        out_shape=jax.ShapeDtypeStruct((BS, D), jnp.bfloat16),
        in_specs=[pl.BlockSpec(memory_space=pltpu.VMEM)],
        out_specs=pl.BlockSpec(memory_space=pltpu.VMEM),
    )(x2)

    q, qr, kr, k, v = pl.pallas_call(
        _proj_body,
        out_shape=[
            jax.ShapeDtypeStruct((BS, D), jnp.bfloat16),
            jax.ShapeDtypeStruct((B * H * S, Dr), jnp.bfloat16),
            jax.ShapeDtypeStruct((BS, Dr), jnp.bfloat16),
            jax.ShapeDtypeStruct((BS, D), jnp.bfloat16),
            jax.ShapeDtypeStruct((BS, D), jnp.bfloat16),
        ],
        in_specs=[pl.BlockSpec(memory_space=pltpu.VMEM)] * 7,
        out_specs=[pl.BlockSpec(memory_space=pltpu.VMEM)] * 5,
        scratch_shapes=[
            pltpu.VMEM((BS, DC), jnp.bfloat16),
            pltpu.VMEM((BS, DC), jnp.bfloat16),
            pltpu.VMEM((DC, D), jnp.bfloat16),
            pltpu.VMEM((DC, D), jnp.bfloat16),
            pltpu.VMEM((DC, D), jnp.bfloat16),
            pltpu.VMEM((DC, D), jnp.bfloat16),
            pltpu.SemaphoreType.DMA((3,)),
            pltpu.SemaphoreType.DMA((3,)),
        ],
        compiler_params=pltpu.CompilerParams(
            collective_id=0, vmem_limit_bytes=60 * 1024 * 1024),
    )(xbf, Wdkv, Wuk, Wuv, Wq, Wqr, Wkr)

    out = pl.pallas_call(
        _attn_out_body,
        grid=(B,),
        out_shape=jax.ShapeDtypeStruct((BS, D), jnp.float32),
        in_specs=[
            pl.BlockSpec((S, D), lambda b: (b, 0)),
            pl.BlockSpec((S, D), lambda b: (b, 0)),
            pl.BlockSpec((S, D), lambda b: (b, 0)),
            pl.BlockSpec((H * S, Dr), lambda b: (b, 0)),
            pl.BlockSpec((S, Dr), lambda b: (b, 0)),
            pl.BlockSpec((D, D), lambda b: (0, 0)),
        ],
        out_specs=pl.BlockSpec((S, D), lambda b: (b, 0)),
        scratch_shapes=[
            pltpu.VMEM((S, D), jnp.bfloat16),
            pltpu.VMEM((D, D), jnp.bfloat16),
        ],
        compiler_params=pltpu.CompilerParams(
            vmem_limit_bytes=60 * 1024 * 1024),
    )(q, k, v, qr, kr, Wo)

    return out.reshape(B, S, D)


# device time: 74616 ns/iter; 1.7184x vs baseline; 1.1268x over previous
import jax
import jax.numpy as jnp
from jax import lax
from jax.experimental import pallas as pl
from jax.experimental.pallas import tpu as pltpu

B, S, H, Dh, Dr = 2, 512, 16, 128, 32
D = 2048
DC = 128
BS = B * S
SCALE = (Dh + Dr) ** -0.5
WQ_CHUNK = 512
N_CHUNKS = D // WQ_CHUNK


def _mm(a, b):
    return lax.dot_general(a, b, (((1,), (0,)), ((), ())),
                           preferred_element_type=jnp.float32)


def _mm_t(a, b):
    return lax.dot_general(a, b, (((1,), (1,)), ((), ())),
                           preferred_element_type=jnp.float32)


def _proj_body(x_ref, wdkv_ref, wuk_ref, wuv_ref, wq_hbm, wqr_ref, wkr_ref,
               q_ref, qr_ref, kr_ref, k_ref, v_ref,
               xbf, c_mine, c_other, wuk_mine, wuk_other, wuv_mine, wuv_other,
               wq_buf, copy_sems, send_sems, recv_sems):
    my_x = lax.axis_index("x")
    my_y = lax.axis_index("y")
    peer = (my_x, 1 - my_y)

    barrier = pltpu.get_barrier_semaphore()
    pl.semaphore_signal(barrier, inc=1, device_id=peer,
                        device_id_type=pl.DeviceIdType.MESH)

    wq_copies = [
        pltpu.make_async_copy(
            wq_hbm.at[:, j * WQ_CHUNK:(j + 1) * WQ_CHUNK],
            wq_buf.at[j % 2], copy_sems.at[j % 2])
        for j in range(N_CHUNKS)
    ]
    wq_copies[0].start()

    xbf[...] = x_ref[...].astype(jnp.bfloat16)
    wuk_mine[...] = wuk_ref[...].astype(jnp.bfloat16)
    wuv_mine[...] = wuv_ref[...].astype(jnp.bfloat16)
    c_mine[...] = _mm(xbf[...],
                      wdkv_ref[...].astype(jnp.bfloat16)).astype(jnp.bfloat16)

    pl.semaphore_wait(barrier, 1)
    copies = []
    for i, (src, dst) in enumerate([(c_mine, c_other),
                                    (wuk_mine, wuk_other),
                                    (wuv_mine, wuv_other)]):
        rdma = pltpu.make_async_remote_copy(
            src_ref=src, dst_ref=dst,
            send_sem=send_sems.at[i], recv_sem=recv_sems.at[i],
            device_id=peer, device_id_type=pl.DeviceIdType.MESH)
        rdma.start()
        copies.append(rdma)

    xb = xbf[...]
    for j in range(N_CHUNKS):
        wq_copies[j].wait()
        if j + 1 < N_CHUNKS:
            wq_copies[j + 1].start()
        q_ref[:, j * WQ_CHUNK:(j + 1) * WQ_CHUNK] = _mm(
            xb, (wq_buf[j % 2] * SCALE).astype(jnp.bfloat16)
        ).astype(jnp.bfloat16)

    qr = _mm(xb, (wqr_ref[...] * SCALE).astype(jnp.bfloat16)).astype(jnp.bfloat16)
    for b in range(B):
        for h in range(H):
            qr_ref[(b * H + h) * S:(b * H + h + 1) * S, :] = (
                qr[b * S:(b + 1) * S, h * Dr:(h + 1) * Dr])
    kr_ref[...] = _mm(xb, wkr_ref[...].astype(jnp.bfloat16)).astype(jnp.bfloat16)

    for rdma in copies:
        rdma.wait()

    half = D // 2
    for j in range(2):
        js = slice(j * half, (j + 1) * half)
        k_ref[:, js] = (_mm(c_mine[...], wuk_mine[:, js])
                        + _mm(c_other[...], wuk_other[:, js])).astype(jnp.bfloat16)
        v_ref[:, js] = (_mm(c_mine[...], wuv_mine[:, js])
                        + _mm(c_other[...], wuv_other[:, js])).astype(jnp.bfloat16)


def _attn_out_body(q_ref, k_ref, v_ref, qr_ref, kr_ref, wo_hbm, out_ref,
                   o_acc, wo_f32, wo_bf, copy_sem):
    b = pl.program_id(0)

    wo_copy = pltpu.make_async_copy(wo_hbm, wo_f32, copy_sem)

    @pl.when(b == 0)
    def _():
        wo_copy.start()

    kr = kr_ref[...]
    for h in range(H):
        hs = slice(h * Dh, (h + 1) * Dh)
        s = _mm_t(q_ref[:, hs], k_ref[:, hs])
        s += _mm_t(qr_ref[h * S:(h + 1) * S, :], kr)
        p = jnp.exp(s)
        rs = jnp.sum(p, axis=1, keepdims=True)
        o = _mm(p, v_ref[:, hs].astype(jnp.float32))
        o_acc[:, hs] = (o * (1.0 / rs)).astype(jnp.bfloat16)

    @pl.when(b == 0)
    def _():
        wo_copy.wait()
        wo_bf[...] = wo_f32[...].astype(jnp.bfloat16)

    half = D // 2
    for j in range(2):
        js = slice(j * half, (j + 1) * half)
        out_ref[:, js] = _mm(o_acc[...], wo_bf[:, js])


def kernel(x, Wdkv, Wuk, Wuv, Wq, Wqr, Wkr, Wo):
    x2 = x.reshape(BS, D)

    q, qr, kr, k, v = pl.pallas_call(
        _proj_body,
        out_shape=[
            jax.ShapeDtypeStruct((BS, D), jnp.bfloat16),
            jax.ShapeDtypeStruct((B * H * S, Dr), jnp.bfloat16),
            jax.ShapeDtypeStruct((BS, Dr), jnp.bfloat16),
            jax.ShapeDtypeStruct((BS, D), jnp.bfloat16),
            jax.ShapeDtypeStruct((BS, D), jnp.bfloat16),
        ],
        in_specs=[
            pl.BlockSpec(memory_space=pltpu.VMEM),
            pl.BlockSpec(memory_space=pltpu.VMEM),
            pl.BlockSpec(memory_space=pltpu.VMEM),
            pl.BlockSpec(memory_space=pltpu.VMEM),
            pl.BlockSpec(memory_space=pl.ANY),
            pl.BlockSpec(memory_space=pltpu.VMEM),
            pl.BlockSpec(memory_space=pltpu.VMEM),
        ],
        out_specs=[pl.BlockSpec(memory_space=pltpu.VMEM)] * 5,
        scratch_shapes=[
            pltpu.VMEM((BS, D), jnp.bfloat16),
            pltpu.VMEM((BS, DC), jnp.bfloat16),
            pltpu.VMEM((BS, DC), jnp.bfloat16),
            pltpu.VMEM((DC, D), jnp.bfloat16),
            pltpu.VMEM((DC, D), jnp.bfloat16),
            pltpu.VMEM((DC, D), jnp.bfloat16),
            pltpu.VMEM((DC, D), jnp.bfloat16),
            pltpu.VMEM((2, D, WQ_CHUNK), jnp.float32),
            pltpu.SemaphoreType.DMA((2,)),
            pltpu.SemaphoreType.DMA((3,)),
            pltpu.SemaphoreType.DMA((3,)),
        ],
        compiler_params=pltpu.CompilerParams(
            collective_id=0, vmem_limit_bytes=60 * 1024 * 1024),
    )(x2, Wdkv, Wuk, Wuv, Wq, Wqr, Wkr)

    out = pl.pallas_call(
        _attn_out_body,
        grid=(B,),
        out_shape=jax.ShapeDtypeStruct((BS, D), jnp.float32),
        in_specs=[
            pl.BlockSpec((S, D), lambda b: (b, 0)),
            pl.BlockSpec((S, D), lambda b: (b, 0)),
            pl.BlockSpec((S, D), lambda b: (b, 0)),
            pl.BlockSpec((H * S, Dr), lambda b: (b, 0)),
            pl.BlockSpec((S, Dr), lambda b: (b, 0)),
            pl.BlockSpec(memory_space=pl.ANY),
        ],
        out_specs=pl.BlockSpec((S, D), lambda b: (b, 0)),
        scratch_shapes=[
            pltpu.VMEM((S, D), jnp.bfloat16),
            pltpu.VMEM((D, D), jnp.float32),
            pltpu.VMEM((D, D), jnp.bfloat16),
            pltpu.SemaphoreType.DMA,
        ],
        compiler_params=pltpu.CompilerParams(
            vmem_limit_bytes=60 * 1024 * 1024),
    )(q, k, v, qr, kr, Wo)

    return out.reshape(B, S, D)
